# Initial kernel scaffold; baseline (speedup 1.0000x reference)
import jax
import jax.numpy as jnp
from jax.experimental import pallas as pl


def kernel(x, indices, values, bias):
    raise NotImplementedError("write your pallas kernel here")


# speedup vs baseline: 20.9710x; 20.9710x over previous
import functools

import jax
import jax.numpy as jnp
from jax import lax
from jax.experimental import pallas as pl
from jax.experimental.pallas import tpu as pltpu
from jax.experimental.pallas import tpu_sc as plsc

L = 16
NC = 2
NS = 16
NW = NC * NS
R = 2
CHUNK = 8192


def _sc_partials(xs, src, dst, val, bias1d, n_bat, n_in, n_out):
    nbg = n_bat // R
    splits = NW // nbg
    nnz_pad = src.shape[0]
    e_per_split = nnz_pad // splits
    n_chunks = e_per_split // CHUNK

    mesh = plsc.VectorSubcoreMesh(core_axis_name="c", subcore_axis_name="s")

    @functools.partial(
        pl.kernel,
        out_type=jax.ShapeDtypeStruct((splits * n_bat, n_out), jnp.float32),
        mesh=mesh,
        compiler_params=pltpu.CompilerParams(needs_layout_passes=False),
        scratch_types=[
            pltpu.VMEM((n_in,), jnp.float32),
            pltpu.VMEM((n_in,), jnp.float32),
            pltpu.VMEM((n_out,), jnp.float32),
            pltpu.VMEM((n_out,), jnp.float32),
            pltpu.VMEM((CHUNK,), jnp.int32),
            pltpu.VMEM((CHUNK,), jnp.int32),
            pltpu.VMEM((CHUNK,), jnp.float32),
        ],
    )
    def spmm(xs_hbm, src_hbm, dst_hbm, val_hbm, bias_hbm, out_hbm,
             x0, x1, a0, a1, sbuf, dbuf, vbuf):
        wid = lax.axis_index("c") * NS + lax.axis_index("s")
        bg = wid % nbg
        sp = wid // nbg
        row0 = bg * R

        pltpu.sync_copy(xs_hbm.at[row0], x0)
        pltpu.sync_copy(xs_hbm.at[row0 + 1], x1)

        @pl.when(sp == 0)
        def _():
            pltpu.sync_copy(bias_hbm, a0)
            pltpu.sync_copy(bias_hbm, a1)

        @pl.when(sp != 0)
        def _():
            zv = jnp.zeros((L,), jnp.float32)

            def zbody(i, c):
                a0[pl.ds(i * L, L)] = zv
                a1[pl.ds(i * L, L)] = zv
                return c

            lax.fori_loop(0, n_out // L, zbody, 0)

        base_e = sp * e_per_split

        def chunk_body(ci, c):
            off = base_e + ci * CHUNK
            pltpu.sync_copy(src_hbm.at[pl.ds(off, CHUNK)], sbuf)
            pltpu.sync_copy(dst_hbm.at[pl.ds(off, CHUNK)], dbuf)
            pltpu.sync_copy(val_hbm.at[pl.ds(off, CHUNK)], vbuf)

            def grp(g, cc):
                o = g * L
                s_idx = sbuf[pl.ds(o, L)]
                d_idx = dbuf[pl.ds(o, L)]
                v = vbuf[pl.ds(o, L)]
                g0 = plsc.load_gather(x0, [s_idx])
                g1 = plsc.load_gather(x1, [s_idx])
                plsc.addupdate_scatter(a0, [d_idx], g0 * v)
                plsc.addupdate_scatter(a1, [d_idx], g1 * v)
                return cc

            lax.fori_loop(0, CHUNK // L, grp, 0)
            return c

        lax.fori_loop(0, n_chunks, chunk_body, 0)

        pltpu.sync_copy(a0, out_hbm.at[sp * n_bat + row0])
        pltpu.sync_copy(a1, out_hbm.at[sp * n_bat + row0 + 1])

    return spmm(xs, src, dst, val, bias1d)


def _tc_reduce(partials, splits, n_bat, n_out):
    blk = 2048

    def body(p_ref, o_ref):
        o_ref[...] = jnp.sum(p_ref[...], axis=0)

    return pl.pallas_call(
        body,
        grid=(n_out // blk,),
        in_specs=[pl.BlockSpec((splits, n_bat, blk), lambda i: (0, 0, i))],
        out_specs=pl.BlockSpec((n_bat, blk), lambda i: (0, i)),
        out_shape=jax.ShapeDtypeStruct((n_bat, n_out), jnp.float32),
    )(partials)


def kernel(x, indices, values, bias):
    n_bat, n_in = x.shape[0], x.shape[1]
    n_out = bias.shape[0]
    nnz = values.shape[0]

    nbg = n_bat // R
    splits = NW // nbg
    gran = splits * CHUNK
    nnz_pad = ((nnz + gran - 1) // gran) * gran

    xs = x[..., 0]
    pad = nnz_pad - nnz
    src = jnp.pad(indices[0], (0, pad))
    dst = jnp.pad(indices[1], (0, pad))
    val = jnp.pad(values, (0, pad))
    bias1d = bias[:, 0]

    partials = _sc_partials(xs, src, dst, val, bias1d, n_bat, n_in, n_out)
    partials = partials.reshape(splits, n_bat, n_out)
    out = _tc_reduce(partials, splits, n_bat, n_out)
    return out[..., None]


# speedup vs baseline: 30.2264x; 1.4413x over previous
import functools

import jax
import jax.numpy as jnp
from jax import lax
from jax.experimental import pallas as pl
from jax.experimental.pallas import tpu as pltpu
from jax.experimental.pallas import tpu_sc as plsc

L = 16
NC = 2
NS = 16
NW = NC * NS
R = 2
CHUNK = 8192


def _sc_partials(xs, src, dst, val, bias1d, n_bat, n_in, n_out):
    nbg = n_bat // R
    splits = NW // nbg
    nnz_pad = src.shape[0]
    e_per_split = nnz_pad // splits
    n_chunks = e_per_split // CHUNK

    mesh = plsc.VectorSubcoreMesh(core_axis_name="c", subcore_axis_name="s")

    @functools.partial(
        pl.kernel,
        out_type=jax.ShapeDtypeStruct((splits * n_bat, n_out), jnp.float32),
        mesh=mesh,
        compiler_params=pltpu.CompilerParams(needs_layout_passes=False),
        scratch_types=[
            pltpu.VMEM((n_in,), jnp.float32),
            pltpu.VMEM((n_in,), jnp.float32),
            pltpu.VMEM((n_out,), jnp.float32),
            pltpu.VMEM((n_out,), jnp.float32),
            pltpu.VMEM((CHUNK,), jnp.int32),
            pltpu.VMEM((CHUNK,), jnp.int32),
            pltpu.VMEM((CHUNK,), jnp.float32),
        ],
    )
    def spmm(xs_hbm, src_hbm, dst_hbm, val_hbm, bias_hbm, out_hbm,
             x0, x1, a0, a1, sbuf, dbuf, vbuf):
        wid = lax.axis_index("c") * NS + lax.axis_index("s")
        bg = wid % nbg
        sp = wid // nbg
        row0 = bg * R

        pltpu.sync_copy(xs_hbm.at[row0], x0)
        pltpu.sync_copy(xs_hbm.at[row0 + 1], x1)

        @pl.when(sp == 0)
        def _():
            pltpu.sync_copy(bias_hbm, a0)
            pltpu.sync_copy(bias_hbm, a1)

        @pl.when(sp != 0)
        def _():
            zv = jnp.zeros((L,), jnp.float32)

            def zbody(i, c):
                a0[pl.ds(i * L, L)] = zv
                a1[pl.ds(i * L, L)] = zv
                return c

            lax.fori_loop(0, n_out // L, zbody, 0)

        base_e = sp * e_per_split

        def chunk_body(ci, c):
            off = base_e + ci * CHUNK
            pltpu.sync_copy(src_hbm.at[pl.ds(off, CHUNK)], sbuf)
            pltpu.sync_copy(dst_hbm.at[pl.ds(off, CHUNK)], dbuf)
            pltpu.sync_copy(val_hbm.at[pl.ds(off, CHUNK)], vbuf)

            @plsc.parallel_loop(0, CHUNK, step=L, unroll=8)
            def grp(o):
                s_idx = sbuf[pl.ds(o, L)]
                d_idx = dbuf[pl.ds(o, L)]
                v = vbuf[pl.ds(o, L)]
                g0 = plsc.load_gather(x0, [s_idx])
                g1 = plsc.load_gather(x1, [s_idx])
                plsc.addupdate_scatter(a0, [d_idx], g0 * v)
                plsc.addupdate_scatter(a1, [d_idx], g1 * v)

            return c

        lax.fori_loop(0, n_chunks, chunk_body, 0)

        pltpu.sync_copy(a0, out_hbm.at[sp * n_bat + row0])
        pltpu.sync_copy(a1, out_hbm.at[sp * n_bat + row0 + 1])

    return spmm(xs, src, dst, val, bias1d)


def _tc_reduce(partials, splits, n_bat, n_out):
    blk = 2048

    def body(p_ref, o_ref):
        o_ref[...] = jnp.sum(p_ref[...], axis=0)

    return pl.pallas_call(
        body,
        grid=(n_out // blk,),
        in_specs=[pl.BlockSpec((splits, n_bat, blk), lambda i: (0, 0, i))],
        out_specs=pl.BlockSpec((n_bat, blk), lambda i: (0, i)),
        out_shape=jax.ShapeDtypeStruct((n_bat, n_out), jnp.float32),
    )(partials)


def kernel(x, indices, values, bias):
    n_bat, n_in = x.shape[0], x.shape[1]
    n_out = bias.shape[0]
    nnz = values.shape[0]

    nbg = n_bat // R
    splits = NW // nbg
    gran = splits * CHUNK
    nnz_pad = ((nnz + gran - 1) // gran) * gran

    xs = x[..., 0]
    pad = nnz_pad - nnz
    src = jnp.pad(indices[0], (0, pad))
    dst = jnp.pad(indices[1], (0, pad))
    val = jnp.pad(values, (0, pad))
    bias1d = bias[:, 0]

    partials = _sc_partials(xs, src, dst, val, bias1d, n_bat, n_in, n_out)
    partials = partials.reshape(splits, n_bat, n_out)
    out = _tc_reduce(partials, splits, n_bat, n_out)
    return out[..., None]


# speedup vs baseline: 45.7526x; 1.5137x over previous
import functools

import jax
import jax.numpy as jnp
from jax import lax
from jax.experimental import pallas as pl
from jax.experimental.pallas import tpu as pltpu
from jax.experimental.pallas import tpu_sc as plsc

L = 16
NC = 2
NS = 16
NW = NC * NS
R = 2
CHUNK = 8192


def _sc_partials(xs, src, dst, val, bias1d, n_bat, n_in, n_out):
    nbg = n_bat // R
    splits = NW // nbg
    nnz_pad = src.shape[0]
    e_per_split = nnz_pad // splits
    n_chunks = e_per_split // CHUNK

    mesh = plsc.VectorSubcoreMesh(core_axis_name="c", subcore_axis_name="s")

    @functools.partial(
        pl.kernel,
        out_type=jax.ShapeDtypeStruct((splits * n_bat, n_out), jnp.float32),
        mesh=mesh,
        compiler_params=pltpu.CompilerParams(needs_layout_passes=False),
        scratch_types=[
            pltpu.VMEM((n_in,), jnp.float32),
            pltpu.VMEM((n_in,), jnp.float32),
            pltpu.VMEM((n_out,), jnp.float32),
            pltpu.VMEM((n_out,), jnp.float32),
            pltpu.VMEM((CHUNK,), jnp.int32),
            pltpu.VMEM((CHUNK,), jnp.int32),
            pltpu.VMEM((CHUNK,), jnp.float32),
            pltpu.VMEM((CHUNK,), jnp.int32),
            pltpu.VMEM((CHUNK,), jnp.int32),
            pltpu.VMEM((CHUNK,), jnp.float32),
            pltpu.SemaphoreType.DMA,
            pltpu.SemaphoreType.DMA,
        ],
    )
    def spmm(xs_hbm, src_hbm, dst_hbm, val_hbm, bias_hbm, out_hbm,
             x0, x1, a0, a1, sb0, db0, vb0, sb1, db1, vb1, sem0, sem1):
        sbufs, dbufs, vbufs, sems = (sb0, sb1), (db0, db1), (vb0, vb1), (sem0, sem1)
        wid = lax.axis_index("c") * NS + lax.axis_index("s")
        bg = wid % nbg
        sp = wid // nbg
        row0 = bg * R

        pltpu.sync_copy(xs_hbm.at[row0], x0)
        pltpu.sync_copy(xs_hbm.at[row0 + 1], x1)

        @pl.when(sp == 0)
        def _():
            pltpu.sync_copy(bias_hbm, a0)
            pltpu.sync_copy(bias_hbm, a1)

        @pl.when(sp != 0)
        def _():
            zv = jnp.zeros((L,), jnp.float32)

            def zbody(i, c):
                a0[pl.ds(i * L, L)] = zv
                a1[pl.ds(i * L, L)] = zv
                return c

            lax.fori_loop(0, n_out // L, zbody, 0)

        base_e = sp * e_per_split

        def start(ci, b):
            off = base_e + ci * CHUNK
            pltpu.async_copy(src_hbm.at[pl.ds(off, CHUNK)], sbufs[b], sems[b])
            pltpu.async_copy(dst_hbm.at[pl.ds(off, CHUNK)], dbufs[b], sems[b])
            pltpu.async_copy(val_hbm.at[pl.ds(off, CHUNK)], vbufs[b], sems[b])

        def wait(ci, b):
            off = base_e + ci * CHUNK
            pltpu.make_async_copy(src_hbm.at[pl.ds(off, CHUNK)], sbufs[b], sems[b]).wait()
            pltpu.make_async_copy(dst_hbm.at[pl.ds(off, CHUNK)], dbufs[b], sems[b]).wait()
            pltpu.make_async_copy(val_hbm.at[pl.ds(off, CHUNK)], vbufs[b], sems[b]).wait()

        def compute(b):
            sbuf, dbuf, vbuf = sbufs[b], dbufs[b], vbufs[b]

            @plsc.parallel_loop(0, CHUNK, step=L, unroll=8)
            def grp(o):
                s_idx = sbuf[pl.ds(o, L)]
                d_idx = dbuf[pl.ds(o, L)]
                v = vbuf[pl.ds(o, L)]
                g0 = plsc.load_gather(x0, [s_idx])
                g1 = plsc.load_gather(x1, [s_idx])
                plsc.addupdate_scatter(a0, [d_idx], g0 * v)
                plsc.addupdate_scatter(a1, [d_idx], g1 * v)

        start(0, 0)

        def chunk_pair(k, c):
            ci = k * 2
            start(ci + 1, 1)
            wait(ci, 0)
            compute(0)

            @pl.when(ci + 2 < n_chunks)
            def _():
                start(ci + 2, 0)

            wait(ci + 1, 1)
            compute(1)
            return c

        lax.fori_loop(0, n_chunks // 2, chunk_pair, 0)

        pltpu.sync_copy(a0, out_hbm.at[sp * n_bat + row0])
        pltpu.sync_copy(a1, out_hbm.at[sp * n_bat + row0 + 1])

    return spmm(xs, src, dst, val, bias1d)


def _tc_reduce(partials, splits, n_bat, n_out):
    blk = 2048

    def body(p_ref, o_ref):
        o_ref[...] = jnp.sum(p_ref[...], axis=0)

    return pl.pallas_call(
        body,
        grid=(n_out // blk,),
        in_specs=[pl.BlockSpec((splits, n_bat, blk), lambda i: (0, 0, i))],
        out_specs=pl.BlockSpec((n_bat, blk), lambda i: (0, i)),
        out_shape=jax.ShapeDtypeStruct((n_bat, n_out), jnp.float32),
    )(partials)


def kernel(x, indices, values, bias):
    n_bat, n_in = x.shape[0], x.shape[1]
    n_out = bias.shape[0]
    nnz = values.shape[0]

    nbg = n_bat // R
    splits = NW // nbg
    gran = splits * CHUNK * 2
    nnz_pad = ((nnz + gran - 1) // gran) * gran

    xs = x[..., 0]
    pad = nnz_pad - nnz
    src = jnp.pad(indices[0], (0, pad))
    dst = jnp.pad(indices[1], (0, pad))
    val = jnp.pad(values, (0, pad))
    bias1d = bias[:, 0]

    partials = _sc_partials(xs, src, dst, val, bias1d, n_bat, n_in, n_out)
    partials = partials.reshape(splits, n_bat, n_out)
    out = _tc_reduce(partials, splits, n_bat, n_out)
    return out[..., None]


# speedup vs baseline: 53.2856x; 1.1646x over previous
import functools

import jax
import jax.numpy as jnp
from jax import lax
from jax.experimental import pallas as pl
from jax.experimental.pallas import tpu as pltpu
from jax.experimental.pallas import tpu_sc as plsc

L = 16
NC = 2
NS = 16
NW = NC * NS
R = 4
CHUNK = 4096


def _sc_partials(xp, src, dst, val, bias1d, n_bat, n_in, n_out):
    nbg = n_bat // R
    splits = NW // nbg
    nnz_pad = src.shape[0]
    e_per_split = nnz_pad // splits
    n_chunks = e_per_split // CHUNK

    mesh = plsc.VectorSubcoreMesh(core_axis_name="c", subcore_axis_name="s")

    @functools.partial(
        pl.kernel,
        out_type=jax.ShapeDtypeStruct((splits * n_bat, n_out), jnp.float32),
        mesh=mesh,
        compiler_params=pltpu.CompilerParams(needs_layout_passes=False),
        scratch_types=[
            pltpu.VMEM((n_in,), jnp.int32),
            pltpu.VMEM((n_in,), jnp.int32),
            pltpu.VMEM((n_out,), jnp.float32),
            pltpu.VMEM((n_out,), jnp.float32),
            pltpu.VMEM((n_out,), jnp.float32),
            pltpu.VMEM((n_out,), jnp.float32),
            pltpu.VMEM((CHUNK,), jnp.int32),
            pltpu.VMEM((CHUNK,), jnp.int32),
            pltpu.VMEM((CHUNK,), jnp.float32),
            pltpu.VMEM((CHUNK,), jnp.int32),
            pltpu.VMEM((CHUNK,), jnp.int32),
            pltpu.VMEM((CHUNK,), jnp.float32),
            pltpu.SemaphoreType.DMA,
            pltpu.SemaphoreType.DMA,
        ],
    )
    def spmm(xp_hbm, src_hbm, dst_hbm, val_hbm, bias_hbm, out_hbm,
             x01, x23, a0, a1, a2, a3, sb0, db0, vb0, sb1, db1, vb1,
             sem0, sem1):
        sbufs, dbufs, vbufs, sems = (sb0, sb1), (db0, db1), (vb0, vb1), (sem0, sem1)
        accs = (a0, a1, a2, a3)
        wid = lax.axis_index("c") * NS + lax.axis_index("s")
        bg = wid % nbg
        sp = wid // nbg
        row0 = bg * R

        pltpu.sync_copy(xp_hbm.at[bg * 2], x01)
        pltpu.sync_copy(xp_hbm.at[bg * 2 + 1], x23)

        @pl.when(sp == 0)
        def _():
            for a in accs:
                pltpu.sync_copy(bias_hbm, a)

        @pl.when(sp != 0)
        def _():
            zv = jnp.zeros((L,), jnp.float32)

            @plsc.parallel_loop(0, n_out, step=L, unroll=4)
            def zbody(o):
                for a in accs:
                    a[pl.ds(o, L)] = zv

        base_e = sp * e_per_split

        def start(ci, b):
            off = base_e + ci * CHUNK
            pltpu.async_copy(src_hbm.at[pl.ds(off, CHUNK)], sbufs[b], sems[b])
            pltpu.async_copy(dst_hbm.at[pl.ds(off, CHUNK)], dbufs[b], sems[b])
            pltpu.async_copy(val_hbm.at[pl.ds(off, CHUNK)], vbufs[b], sems[b])

        def wait(ci, b):
            off = base_e + ci * CHUNK
            pltpu.make_async_copy(src_hbm.at[pl.ds(off, CHUNK)], sbufs[b], sems[b]).wait()
            pltpu.make_async_copy(dst_hbm.at[pl.ds(off, CHUNK)], dbufs[b], sems[b]).wait()
            pltpu.make_async_copy(val_hbm.at[pl.ds(off, CHUNK)], vbufs[b], sems[b]).wait()

        hi_mask = jnp.full((L,), -65536, jnp.int32)

        def compute(b):
            sbuf, dbuf, vbuf = sbufs[b], dbufs[b], vbufs[b]

            @plsc.parallel_loop(0, CHUNK, step=L, unroll=8)
            def grp(o):
                s_idx = sbuf[pl.ds(o, L)]
                d_idx = dbuf[pl.ds(o, L)]
                v = vbuf[pl.ds(o, L)]
                g01 = plsc.load_gather(x01, [s_idx])
                g23 = plsc.load_gather(x23, [s_idx])
                r0 = plsc.bitcast(lax.bitwise_and(g01, hi_mask), jnp.float32)
                r1 = plsc.bitcast(lax.shift_left(g01, 16), jnp.float32)
                r2 = plsc.bitcast(lax.bitwise_and(g23, hi_mask), jnp.float32)
                r3 = plsc.bitcast(lax.shift_left(g23, 16), jnp.float32)
                plsc.addupdate_scatter(a0, [d_idx], r0 * v)
                plsc.addupdate_scatter(a1, [d_idx], r1 * v)
                plsc.addupdate_scatter(a2, [d_idx], r2 * v)
                plsc.addupdate_scatter(a3, [d_idx], r3 * v)

        start(0, 0)

        def chunk_pair(k, c):
            ci = k * 2
            start(ci + 1, 1)
            wait(ci, 0)
            compute(0)

            @pl.when(ci + 2 < n_chunks)
            def _():
                start(ci + 2, 0)

            wait(ci + 1, 1)
            compute(1)
            return c

        lax.fori_loop(0, n_chunks // 2, chunk_pair, 0)

        for j, a in enumerate(accs):
            pltpu.sync_copy(a, out_hbm.at[sp * n_bat + row0 + j])

    return spmm(xp, src, dst, val, bias1d)


def _tc_reduce(partials, splits, n_bat, n_out):
    blk = 2048

    def body(p_ref, o_ref):
        o_ref[...] = jnp.sum(p_ref[...], axis=0)

    return pl.pallas_call(
        body,
        grid=(n_out // blk,),
        in_specs=[pl.BlockSpec((splits, n_bat, blk), lambda i: (0, 0, i))],
        out_specs=pl.BlockSpec((n_bat, blk), lambda i: (0, i)),
        out_shape=jax.ShapeDtypeStruct((n_bat, n_out), jnp.float32),
    )(partials)


def kernel(x, indices, values, bias):
    n_bat, n_in = x.shape[0], x.shape[1]
    n_out = bias.shape[0]
    nnz = values.shape[0]

    nbg = n_bat // R
    splits = NW // nbg
    gran = splits * CHUNK * 2
    nnz_pad = ((nnz + gran - 1) // gran) * gran

    xs = x[..., 0]
    xb = jax.lax.convert_element_type(xs, jnp.bfloat16)
    xi = jax.lax.bitcast_convert_type(xb, jnp.uint16).astype(jnp.uint32)
    xpack = jax.lax.bitcast_convert_type(
        (xi[0::2] << 16) | xi[1::2], jnp.int32)

    pad = nnz_pad - nnz
    src = jnp.pad(indices[0], (0, pad))
    dst = jnp.pad(indices[1], (0, pad))
    val = jnp.pad(values, (0, pad))
    bias1d = bias[:, 0]

    partials = _sc_partials(xpack, src, dst, val, bias1d, n_bat, n_in, n_out)
    partials = partials.reshape(splits, n_bat, n_out)
    out = _tc_reduce(partials, splits, n_bat, n_out)
    return out[..., None]


# speedup vs baseline: 54.4333x; 1.0215x over previous
import functools

import jax
import jax.numpy as jnp
from jax import lax
from jax.experimental import pallas as pl
from jax.experimental.pallas import tpu as pltpu
from jax.experimental.pallas import tpu_sc as plsc

L = 16
NC = 2
NS = 16
NW = NC * NS
R = 4
CHUNK = 4096


def _tc_pack(src, dst, val, nnz_pad):
    nnz = src.shape[0]
    blkn = CHUNK * 8
    grid = nnz_pad // blkn

    def body(s_ref, d_ref, v_ref, pk_ref, vp_ref):
        pos = pl.program_id(0) * blkn + lax.broadcasted_iota(
            jnp.int32, (1, blkn), 1)
        keep = pos < nnz
        s = s_ref[...]
        d = d_ref[...]
        pk_ref[...] = jnp.where(keep, (d << 16) | s, 0)
        vp_ref[...] = jnp.where(keep, v_ref[...], 0.0)

    pk, vp = pl.pallas_call(
        body,
        grid=(grid,),
        in_specs=[
            pl.BlockSpec((1, blkn), lambda i: (0, i)),
            pl.BlockSpec((1, blkn), lambda i: (0, i)),
            pl.BlockSpec((1, blkn), lambda i: (0, i)),
        ],
        out_specs=[
            pl.BlockSpec((1, blkn), lambda i: (0, i)),
            pl.BlockSpec((1, blkn), lambda i: (0, i)),
        ],
        out_shape=[
            jax.ShapeDtypeStruct((1, nnz_pad), jnp.int32),
            jax.ShapeDtypeStruct((1, nnz_pad), jnp.float32),
        ],
    )(src.reshape(1, nnz), dst.reshape(1, nnz), val.reshape(1, nnz))
    return pk.reshape(nnz_pad), vp.reshape(nnz_pad)


def _sc_partials(xp, pidx, val, bias1d, n_bat, n_in, n_out):
    nbg = n_bat // R
    splits = NW // nbg
    nnz_pad = pidx.shape[0]
    e_per_split = nnz_pad // splits
    n_chunks = e_per_split // CHUNK

    mesh = plsc.VectorSubcoreMesh(core_axis_name="c", subcore_axis_name="s")

    @functools.partial(
        pl.kernel,
        out_type=jax.ShapeDtypeStruct((splits * n_bat, n_out), jnp.float32),
        mesh=mesh,
        compiler_params=pltpu.CompilerParams(needs_layout_passes=False),
        scratch_types=[
            pltpu.VMEM((n_in,), jnp.int32),
            pltpu.VMEM((n_in,), jnp.int32),
            pltpu.VMEM((n_out,), jnp.float32),
            pltpu.VMEM((n_out,), jnp.float32),
            pltpu.VMEM((n_out,), jnp.float32),
            pltpu.VMEM((n_out,), jnp.float32),
            pltpu.VMEM((CHUNK,), jnp.int32),
            pltpu.VMEM((CHUNK,), jnp.float32),
            pltpu.VMEM((CHUNK,), jnp.int32),
            pltpu.VMEM((CHUNK,), jnp.float32),
            pltpu.SemaphoreType.DMA,
            pltpu.SemaphoreType.DMA,
        ],
    )
    def spmm(xp_hbm, pidx_hbm, val_hbm, bias_hbm, out_hbm,
             x01, x23, a0, a1, a2, a3, pb0, vb0, pb1, vb1, sem0, sem1):
        pbufs, vbufs, sems = (pb0, pb1), (vb0, vb1), (sem0, sem1)
        accs = (a0, a1, a2, a3)
        wid = lax.axis_index("c") * NS + lax.axis_index("s")
        bg = wid % nbg
        sp = wid // nbg
        row0 = bg * R

        pltpu.sync_copy(xp_hbm.at[bg * 2], x01)
        pltpu.sync_copy(xp_hbm.at[bg * 2 + 1], x23)

        @pl.when(sp == 0)
        def _():
            for a in accs:
                pltpu.sync_copy(bias_hbm, a)

        @pl.when(sp != 0)
        def _():
            zv = jnp.zeros((L,), jnp.float32)

            @plsc.parallel_loop(0, n_out, step=L, unroll=4)
            def zbody(o):
                for a in accs:
                    a[pl.ds(o, L)] = zv

        base_e = sp * e_per_split

        def start(ci, b):
            off = base_e + ci * CHUNK
            pltpu.async_copy(pidx_hbm.at[pl.ds(off, CHUNK)], pbufs[b], sems[b])
            pltpu.async_copy(val_hbm.at[pl.ds(off, CHUNK)], vbufs[b], sems[b])

        def wait(ci, b):
            off = base_e + ci * CHUNK
            pltpu.make_async_copy(pidx_hbm.at[pl.ds(off, CHUNK)], pbufs[b], sems[b]).wait()
            pltpu.make_async_copy(val_hbm.at[pl.ds(off, CHUNK)], vbufs[b], sems[b]).wait()

        hi_mask = jnp.full((L,), -65536, jnp.int32)
        lo_mask = jnp.full((L,), 65535, jnp.int32)

        def compute(b):
            pbuf, vbuf = pbufs[b], vbufs[b]

            @plsc.parallel_loop(0, CHUNK, step=L, unroll=8)
            def grp(o):
                p = pbuf[pl.ds(o, L)]
                v = vbuf[pl.ds(o, L)]
                s_idx = lax.bitwise_and(p, lo_mask)
                d_idx = lax.shift_right_logical(p, 16)
                g01 = plsc.load_gather(x01, [s_idx])
                g23 = plsc.load_gather(x23, [s_idx])
                r0 = plsc.bitcast(lax.bitwise_and(g01, hi_mask), jnp.float32)
                r1 = plsc.bitcast(lax.shift_left(g01, 16), jnp.float32)
                r2 = plsc.bitcast(lax.bitwise_and(g23, hi_mask), jnp.float32)
                r3 = plsc.bitcast(lax.shift_left(g23, 16), jnp.float32)
                plsc.addupdate_scatter(a0, [d_idx], r0 * v)
                plsc.addupdate_scatter(a1, [d_idx], r1 * v)
                plsc.addupdate_scatter(a2, [d_idx], r2 * v)
                plsc.addupdate_scatter(a3, [d_idx], r3 * v)

        start(0, 0)

        def chunk_pair(k, c):
            ci = k * 2
            start(ci + 1, 1)
            wait(ci, 0)
            compute(0)

            @pl.when(ci + 2 < n_chunks)
            def _():
                start(ci + 2, 0)

            wait(ci + 1, 1)
            compute(1)
            return c

        lax.fori_loop(0, n_chunks // 2, chunk_pair, 0)

        for j, a in enumerate(accs):
            pltpu.sync_copy(a, out_hbm.at[sp * n_bat + row0 + j])

    return spmm(xp, pidx, val, bias1d)


def _tc_reduce(partials, splits, n_bat, n_out):
    blk = 2048

    def body(p_ref, o_ref):
        o_ref[...] = jnp.sum(p_ref[...], axis=0)

    return pl.pallas_call(
        body,
        grid=(n_out // blk,),
        in_specs=[pl.BlockSpec((splits, n_bat, blk), lambda i: (0, 0, i))],
        out_specs=pl.BlockSpec((n_bat, blk), lambda i: (0, i)),
        out_shape=jax.ShapeDtypeStruct((n_bat, n_out), jnp.float32),
    )(partials)


def kernel(x, indices, values, bias):
    n_bat, n_in = x.shape[0], x.shape[1]
    n_out = bias.shape[0]
    nnz = values.shape[0]

    nbg = n_bat // R
    splits = NW // nbg
    gran = splits * CHUNK * 2
    nnz_pad = ((nnz + gran - 1) // gran) * gran

    xs = x[..., 0]
    xb = jax.lax.convert_element_type(xs, jnp.bfloat16)
    xi = jax.lax.bitcast_convert_type(xb, jnp.uint16).astype(jnp.uint32)
    xpack = jax.lax.bitcast_convert_type(
        (xi[0::2] << 16) | xi[1::2], jnp.int32)

    pidx, val = _tc_pack(indices[0], indices[1], values, nnz_pad)
    bias1d = bias[:, 0]

    partials = _sc_partials(xpack, pidx, val, bias1d, n_bat, n_in, n_out)
    partials = partials.reshape(splits, n_bat, n_out)
    out = _tc_reduce(partials, splits, n_bat, n_out)
    return out[..., None]


# speedup vs baseline: 80.5339x; 1.4795x over previous
import functools

import jax
import jax.numpy as jnp
from jax import lax
from jax.experimental import pallas as pl
from jax.experimental.pallas import tpu as pltpu
from jax.experimental.pallas import tpu_sc as plsc

L = 16
NC = 2
NS = 16
NW = NC * NS
R = 4
CHUNK = 4096


def _sc_partials(xs, indices, values, tidx, tval, bias1d, n_bat, n_in, n_out):
    nbg = n_bat // R
    splits = NW // nbg
    nnz = values.shape[0]
    n_full = nnz // CHUNK
    q = n_full // splits
    n_pairs = q // 2

    mesh = plsc.VectorSubcoreMesh(core_axis_name="c", subcore_axis_name="s")

    @functools.partial(
        pl.kernel,
        out_type=jax.ShapeDtypeStruct((splits * n_bat, n_out), jnp.float32),
        mesh=mesh,
        compiler_params=pltpu.CompilerParams(needs_layout_passes=False),
        scratch_types=[
            pltpu.VMEM((n_in,), jnp.int32),
            pltpu.VMEM((n_in,), jnp.int32),
            pltpu.VMEM((n_out,), jnp.float32),
            pltpu.VMEM((n_out,), jnp.float32),
            pltpu.VMEM((n_out,), jnp.float32),
            pltpu.VMEM((n_out,), jnp.float32),
            pltpu.VMEM((2, CHUNK), jnp.int32),
            pltpu.VMEM((CHUNK,), jnp.float32),
            pltpu.VMEM((2, CHUNK), jnp.int32),
            pltpu.VMEM((CHUNK,), jnp.float32),
            pltpu.SemaphoreType.DMA,
            pltpu.SemaphoreType.DMA,
        ],
    )
    def spmm(xs_hbm, idx_hbm, val_hbm, tidx_hbm, tval_hbm, bias_hbm, out_hbm,
             x01, x23, a0, a1, a2, a3, ib0, vb0, ib1, vb1, sem0, sem1):
        ibufs, vbufs, sems = (ib0, ib1), (vb0, vb1), (sem0, sem1)
        accs = (a0, a1, a2, a3)
        wid = lax.axis_index("c") * NS + lax.axis_index("s")
        bg = wid % nbg
        sp = wid // nbg
        row0 = bg * R

        half = jnp.full((L,), 0x7FFF, jnp.int32)
        one = jnp.full((L,), 1, jnp.int32)
        hi_mask = jnp.full((L,), -65536, jnp.int32)

        def rnd(f):
            b = plsc.bitcast(f, jnp.int32)
            return b + half + lax.bitwise_and(lax.shift_right_logical(b, 16), one)

        for dst_ref, j in ((x01, 0), (x23, 2)):
            pltpu.sync_copy(xs_hbm.at[row0 + j], a0)
            pltpu.sync_copy(xs_hbm.at[row0 + j + 1], a1)

            @plsc.parallel_loop(0, n_in, step=L, unroll=4)
            def pack(o):
                hi = lax.bitwise_and(rnd(a0[pl.ds(o, L)]), hi_mask)
                lo = lax.shift_right_logical(rnd(a1[pl.ds(o, L)]), 16)
                dst_ref[pl.ds(o, L)] = lax.bitwise_or(hi, lo)

        @pl.when(sp == 0)
        def _():
            for a in accs:
                pltpu.sync_copy(bias_hbm, a)

        @pl.when(sp != 0)
        def _():
            zv = jnp.zeros((L,), jnp.float32)

            @plsc.parallel_loop(0, n_out, step=L, unroll=4)
            def zbody(o):
                for a in accs:
                    a[pl.ds(o, L)] = zv

        def start(ci, b):
            off = (ci * splits + sp) * CHUNK
            pltpu.async_copy(idx_hbm.at[:, pl.ds(off, CHUNK)], ibufs[b], sems[b])
            pltpu.async_copy(val_hbm.at[pl.ds(off, CHUNK)], vbufs[b], sems[b])

        def wait(ci, b):
            off = (ci * splits + sp) * CHUNK
            pltpu.make_async_copy(idx_hbm.at[:, pl.ds(off, CHUNK)], ibufs[b], sems[b]).wait()
            pltpu.make_async_copy(val_hbm.at[pl.ds(off, CHUNK)], vbufs[b], sems[b]).wait()

        def compute(b):
            ibuf, vbuf = ibufs[b], vbufs[b]

            @plsc.parallel_loop(0, CHUNK, step=L, unroll=8)
            def grp(o):
                s_idx = ibuf[0, pl.ds(o, L)]
                d_idx = ibuf[1, pl.ds(o, L)]
                v = vbuf[pl.ds(o, L)]
                g01 = plsc.load_gather(x01, [s_idx])
                g23 = plsc.load_gather(x23, [s_idx])
                r0 = plsc.bitcast(lax.bitwise_and(g01, hi_mask), jnp.float32)
                r1 = plsc.bitcast(lax.shift_left(g01, 16), jnp.float32)
                r2 = plsc.bitcast(lax.bitwise_and(g23, hi_mask), jnp.float32)
                r3 = plsc.bitcast(lax.shift_left(g23, 16), jnp.float32)
                plsc.addupdate_scatter(a0, [d_idx], r0 * v)
                plsc.addupdate_scatter(a1, [d_idx], r1 * v)
                plsc.addupdate_scatter(a2, [d_idx], r2 * v)
                plsc.addupdate_scatter(a3, [d_idx], r3 * v)

        @pl.when(jnp.bool_(n_pairs > 0))
        def _():
            start(0, 0)

            def chunk_pair(k, c):
                ci = k * 2
                start(ci + 1, 1)
                wait(ci, 0)
                compute(0)

                @pl.when(ci + 2 < n_pairs * 2)
                def _():
                    start(ci + 2, 0)

                wait(ci + 1, 1)
                compute(1)
                return c

            lax.fori_loop(0, n_pairs, chunk_pair, 0)

        for j in range(2 * n_pairs, (n_full + splits - 1) // splits):
            @pl.when(j * splits + sp < n_full)
            def _():
                b = j % 2
                start(j, b)
                wait(j, b)
                compute(b)

        @pl.when(sp == splits - 1)
        def _():
            pltpu.async_copy(tidx_hbm, ib0, sem0)
            pltpu.async_copy(tval_hbm, vb0, sem0)
            pltpu.make_async_copy(tidx_hbm, ib0, sem0).wait()
            pltpu.make_async_copy(tval_hbm, vb0, sem0).wait()
            compute(0)

        for j, a in enumerate(accs):
            pltpu.sync_copy(a, out_hbm.at[sp * n_bat + row0 + j])

    return spmm(xs, indices, values, tidx, tval, bias1d)


def _tc_reduce(partials, splits, n_bat, n_out):
    blk = 2048

    def body(p_ref, o_ref):
        o_ref[...] = jnp.sum(p_ref[...], axis=0)

    return pl.pallas_call(
        body,
        grid=(n_out // blk,),
        in_specs=[pl.BlockSpec((splits, n_bat, blk), lambda i: (0, 0, i))],
        out_specs=pl.BlockSpec((n_bat, blk), lambda i: (0, i)),
        out_shape=jax.ShapeDtypeStruct((n_bat, n_out), jnp.float32),
    )(partials)


def kernel(x, indices, values, bias):
    n_bat, n_in = x.shape[0], x.shape[1]
    n_out = bias.shape[0]
    nnz = values.shape[0]

    nbg = n_bat // R
    splits = NW // nbg

    xs = x.reshape(n_bat, n_in)
    bias1d = bias.reshape(n_out)

    n_full = nnz // CHUNK
    t = nnz - n_full * CHUNK
    tidx = jnp.pad(lax.slice(indices, (0, nnz - t), (2, nnz)),
                   ((0, 0), (0, CHUNK - t)))
    tval = jnp.pad(lax.slice(values, (nnz - t,), (nnz,)), (0, CHUNK - t))

    partials = _sc_partials(xs, indices, values, tidx, tval, bias1d,
                            n_bat, n_in, n_out)
    partials = partials.reshape(splits, n_bat, n_out)
    out = _tc_reduce(partials, splits, n_bat, n_out)
    return out[..., None]
